# Initial kernel scaffold; baseline (speedup 1.0000x reference)
import jax
import jax.numpy as jnp
from jax.experimental import pallas as pl


def kernel(x, table):
    raise NotImplementedError("write your pallas kernel here")


# speedup vs baseline: 1.1430x; 1.1430x over previous
import math

import jax
import jax.numpy as jnp
from jax import lax
from jax.experimental import pallas as pl
from jax.experimental.pallas import tpu as pltpu
from jax.experimental.pallas import tpu_sc as plsc

D_MODEL = 512
LANES = 16
NUM_CORES = 2
NUM_SUBCORES = 16
NUM_WORKERS = NUM_CORES * NUM_SUBCORES
SCALE = math.sqrt(D_MODEL)

CHUNK = 64


def _emb_body(idx_hbm, table_hbm, out_hbm, idx_v, buf_v, sem):
    cid = lax.axis_index("c")
    sid = lax.axis_index("s")
    wid = sid * NUM_CORES + cid

    n_chunks = idx_hbm.shape[1]
    cpw = n_chunks * CHUNK

    pltpu.sync_copy(idx_hbm.at[wid], idx_v)

    for ch in range(n_chunks):
        pltpu.async_copy(table_hbm.at[idx_v.at[ch]], buf_v, sem).wait()

        def row_body(r, carry):
            for g in range(D_MODEL // LANES):
                sl = pl.ds(g * LANES, LANES)
                buf_v[r, sl] = buf_v[r, sl] * SCALE
            return carry

        lax.fori_loop(0, CHUNK, row_body, 0)

        pltpu.sync_copy(buf_v, out_hbm.at[pl.ds(wid * cpw + ch * CHUNK, CHUNK)])


def kernel(x, table):
    b, s = x.shape
    n_total = b * s
    assert n_total % (NUM_WORKERS * CHUNK) == 0
    n_chunks = n_total // (NUM_WORKERS * CHUNK)
    idx = x.reshape(NUM_WORKERS, n_chunks, CHUNK).astype(jnp.int32)

    mesh = plsc.VectorSubcoreMesh(
        core_axis_name="c",
        subcore_axis_name="s",
        num_cores=NUM_CORES,
        num_subcores=NUM_SUBCORES,
    )
    out = pl.kernel(
        _emb_body,
        out_type=jax.ShapeDtypeStruct((n_total, D_MODEL), jnp.float32),
        mesh=mesh,
        scratch_types=[
            pltpu.VMEM((n_chunks, CHUNK), jnp.int32),
            pltpu.VMEM((CHUNK, D_MODEL), jnp.float32),
            pltpu.SemaphoreType.DMA,
        ],
    )(idx, table)
    return out.reshape(b, s, D_MODEL)


# speedup vs baseline: 1.4239x; 1.2458x over previous
import math

import jax
import jax.numpy as jnp
from jax import lax
from jax.experimental import pallas as pl
from jax.experimental.pallas import tpu as pltpu
from jax.experimental.pallas import tpu_sc as plsc

D_MODEL = 512
LANES = 16
NUM_CORES = 2
NUM_SUBCORES = 16
NUM_WORKERS = NUM_CORES * NUM_SUBCORES
SCALE = math.sqrt(D_MODEL)

CHUNK = 64


def _scale_rows(buf_v):
    def row_body(r, carry):
        for g in range(D_MODEL // LANES):
            sl = pl.ds(g * LANES, LANES)
            buf_v[r, sl] = buf_v[r, sl] * SCALE
        return carry

    lax.fori_loop(0, buf_v.shape[0], row_body, 0)


def _emb_body(idx_hbm, table_hbm, out_hbm, idx_v, buf0, buf1, g0, g1, o0, o1):
    cid = lax.axis_index("c")
    sid = lax.axis_index("s")
    wid = sid * NUM_CORES + cid

    n_chunks = idx_hbm.shape[1]
    cpw = n_chunks * CHUNK

    pltpu.sync_copy(idx_hbm.at[wid], idx_v)

    bufs = (buf0, buf1)
    gsems = (g0, g1)
    osems = (o0, o1)
    out_cp = [None, None]

    gather_cp = [
        pltpu.async_copy(table_hbm.at[idx_v.at[0]], bufs[0], gsems[0]),
        None,
    ]
    for ch in range(n_chunks):
        b = ch & 1
        nb = (ch + 1) & 1
        if ch + 1 < n_chunks:
            if out_cp[nb] is not None:
                out_cp[nb].wait()
                out_cp[nb] = None
            gather_cp[nb] = pltpu.async_copy(
                table_hbm.at[idx_v.at[ch + 1]], bufs[nb], gsems[nb]
            )
        gather_cp[b].wait()
        _scale_rows(bufs[b])
        out_cp[b] = pltpu.async_copy(
            bufs[b], out_hbm.at[pl.ds(wid * cpw + ch * CHUNK, CHUNK)], osems[b]
        )
    for cp in out_cp:
        if cp is not None:
            cp.wait()


def kernel(x, table):
    b, s = x.shape
    n_total = b * s
    assert n_total % (NUM_WORKERS * CHUNK) == 0
    n_chunks = n_total // (NUM_WORKERS * CHUNK)
    idx = x.reshape(NUM_WORKERS, n_chunks, CHUNK).astype(jnp.int32)

    mesh = plsc.VectorSubcoreMesh(
        core_axis_name="c",
        subcore_axis_name="s",
        num_cores=NUM_CORES,
        num_subcores=NUM_SUBCORES,
    )
    out = pl.kernel(
        _emb_body,
        out_type=jax.ShapeDtypeStruct((n_total, D_MODEL), jnp.float32),
        mesh=mesh,
        scratch_types=[
            pltpu.VMEM((n_chunks, CHUNK), jnp.int32),
            pltpu.VMEM((CHUNK, D_MODEL), jnp.float32),
            pltpu.VMEM((CHUNK, D_MODEL), jnp.float32),
            pltpu.SemaphoreType.DMA,
            pltpu.SemaphoreType.DMA,
            pltpu.SemaphoreType.DMA,
            pltpu.SemaphoreType.DMA,
        ],
    )(idx, table)
    return out.reshape(b, s, D_MODEL)
